# baseline (speedup 1.0000x reference)
import functools
import math

import jax
import jax.numpy as jnp
from jax import lax
from jax.experimental import pallas as pl
from jax.experimental.pallas import tpu as pltpu
from jax.experimental.pallas import tpu_sc as plsc

D = 64
SCALE = math.sqrt(D)
NBUF = 2
ROWS = 128


def _make_sc_lookup(S0: int, S1: int, V: int):
  info = plsc.get_sparse_core_info()
  NC, NS, L = info.num_cores, info.num_subcores, info.num_lanes
  NW = NC * NS
  B = S0 * S1
  n_mblk = S0 // ROWS
  n_groups = S1 * n_mblk
  assert n_groups % NW == 0
  g_per_w = n_groups // NW
  assert g_per_w % NBUF == 0
  d_blk = D // 8

  mesh = plsc.VectorSubcoreMesh(core_axis_name="c", subcore_axis_name="s")

  @functools.partial(
      pl.kernel,
      out_type=jax.ShapeDtypeStruct((S1, d_blk, n_mblk, 8, L * 8), jnp.float32),
      mesh=mesh,
      scratch_types=[
          pltpu.VMEM((g_per_w * ROWS,), jnp.int32),
          pltpu.VMEM((NBUF, ROWS, D), jnp.float32),
          pltpu.VMEM((NBUF, d_blk, 8, L * 8), jnp.float32),
          pltpu.SemaphoreType.DMA,
          pltpu.SemaphoreType.DMA,
          pltpu.SemaphoreType.DMA,
          pltpu.SemaphoreType.DMA,
      ],
      compiler_params=pltpu.CompilerParams(
          use_tc_tiling_on_sc=False, needs_layout_passes=False),
  )
  def lookup(idx_hbm, table_hbm, out_hbm, idx_v, rows_v, tbuf, g0, g1, s0, s1):
    gsem = (g0, g1)
    ssem = (s0, s1)
    wid = lax.axis_index("s") * NC + lax.axis_index("c")
    base = wid * g_per_w
    pltpu.sync_copy(idx_hbm.at[pl.ds(base * ROWS, g_per_w * ROWS)], idx_v)

    def gather(gl, b):
      return pltpu.make_async_copy(
          table_hbm.at[idx_v.at[pl.ds(gl * ROWS, ROWS)]], rows_v.at[b], gsem[b])

    def stores(gl, b):
      gid = base + gl
      j = gid >> 5
      m = gid & (n_mblk - 1)
      return [
          pltpu.make_async_copy(tbuf.at[b, k], out_hbm.at[j, k, m], ssem[b])
          for k in range(d_blk)
      ]

    lane = lax.iota(jnp.int32, L)
    row_ids = [lane + (icg * L) for icg in range(ROWS // L)]

    for b in range(NBUF):
      gather(b, b).start()

    @pl.loop(0, g_per_w, step=NBUF)
    def _(g):
      for b in range(NBUF):
        gl = g + b
        gather(gl, b).wait()

        @pl.when(gl >= NBUF)
        def _():
          for d in stores(gl - NBUF, b):
            d.wait()

        rows_b = rows_v.at[b]

        @plsc.parallel_loop(0, D, unroll=2)
        def _(f):
          col = jnp.full((L,), f, jnp.int32)
          for icg in range(ROWS // L):
            v = plsc.load_gather(rows_b, [row_ids[icg], col])
            tbuf[b, f >> 3, f & 7, pl.ds(icg * L, L)] = v * SCALE

        nxt = gl + NBUF

        @pl.when(nxt < g_per_w)
        def _():
          gather(nxt, b).start()

        for d in stores(gl, b):
          d.start()

    for b in range(NBUF):
      for d in stores(g_per_w - NBUF + b, b):
        d.wait()

  return lookup


def kernel(x, table):
  S0, S1 = x.shape
  V, _ = table.shape
  idx = x.T.reshape(S0 * S1).astype(jnp.int32)
  arr = _make_sc_lookup(S0, S1, V)(idx, table)
  return arr.transpose((2, 4, 0, 1, 3)).reshape(S0, S1, D)


# speedup vs baseline: 1.6255x; 1.6255x over previous
import functools
import math

import jax
import jax.numpy as jnp
from jax import lax
from jax.experimental import pallas as pl
from jax.experimental.pallas import tpu as pltpu
from jax.experimental.pallas import tpu_sc as plsc

D = 64
SCALE = math.sqrt(D)
NBUF = 2
ROWS = 128


def _make_sc_lookup(S0: int, S1: int, V: int):
  info = plsc.get_sparse_core_info()
  NC, NS, L = info.num_cores, info.num_subcores, info.num_lanes
  NW = NC * NS
  B = S0 * S1
  n_mblk = S0 // ROWS
  n_groups = S1 * n_mblk
  assert n_groups % NW == 0
  g_per_w = n_groups // NW
  assert g_per_w % NBUF == 0
  d_blk = D // 8

  mesh = plsc.VectorSubcoreMesh(core_axis_name="c", subcore_axis_name="s")

  @functools.partial(
      pl.kernel,
      out_type=jax.ShapeDtypeStruct((S1, d_blk, n_mblk, 8, L * 8), jnp.float32),
      mesh=mesh,
      scratch_types=[
          pltpu.VMEM((g_per_w * ROWS,), jnp.int32),
          pltpu.VMEM((NBUF, ROWS, D), jnp.float32),
          pltpu.VMEM((NBUF, D, L * 8 + 1), jnp.float32),
          pltpu.SemaphoreType.DMA,
          pltpu.SemaphoreType.DMA,
          pltpu.SemaphoreType.DMA,
          pltpu.SemaphoreType.DMA,
      ],
      compiler_params=pltpu.CompilerParams(
          use_tc_tiling_on_sc=False, needs_layout_passes=False),
  )
  def lookup(idx_hbm, table_hbm, out_hbm, idx_v, rows_v, tbuf, g0, g1, s0, s1):
    gsem = (g0, g1)
    ssem = (s0, s1)
    wid = lax.axis_index("s") * NC + lax.axis_index("c")
    base = wid * g_per_w
    pltpu.sync_copy(idx_hbm.at[pl.ds(base * ROWS, g_per_w * ROWS)], idx_v)

    def gather(gl, b):
      return pltpu.make_async_copy(
          table_hbm.at[idx_v.at[pl.ds(gl * ROWS, ROWS)]], rows_v.at[b], gsem[b])

    def stores(gl, b):
      gid = base + gl
      j = gid >> 5
      m = gid & (n_mblk - 1)
      return [
          pltpu.make_async_copy(
              tbuf.at[b, pl.ds(8 * k, 8), pl.ds(0, L * 8)],
              out_hbm.at[j, k, m], ssem[b])
          for k in range(d_blk)
      ]

    lane = lax.iota(jnp.int32, L)
    f_ids = [lane + (fg * L) for fg in range(D // L)]

    for b in range(NBUF):
      gather(b, b).start()

    @pl.loop(0, g_per_w, step=NBUF)
    def _(g):
      for b in range(NBUF):
        gl = g + b
        gather(gl, b).wait()

        @pl.when(gl >= NBUF)
        def _():
          for d in stores(gl - NBUF, b):
            d.wait()

        tb = tbuf.at[b]

        @plsc.parallel_loop(0, ROWS, unroll=4)
        def _(ic):
          col = jnp.full((L,), ic, jnp.int32)
          for fg in range(D // L):
            v = rows_v[b, ic, pl.ds(fg * L, L)]
            plsc.store_scatter(tb, [f_ids[fg], col], v * SCALE)

        nxt = gl + NBUF

        @pl.when(nxt < g_per_w)
        def _():
          gather(nxt, b).start()

        for d in stores(gl, b):
          d.start()

    for b in range(NBUF):
      for d in stores(g_per_w - NBUF + b, b):
        d.wait()

  return lookup


def kernel(x, table):
  S0, S1 = x.shape
  V, _ = table.shape
  idx = x.T.reshape(S0 * S1).astype(jnp.int32)
  arr = _make_sc_lookup(S0, S1, V)(idx, table)
  return arr.transpose((2, 4, 0, 1, 3)).reshape(S0, S1, D)
